# Initial kernel scaffold; baseline (speedup 1.0000x reference)
import jax
import jax.numpy as jnp
from jax.experimental import pallas as pl


def kernel(edge_index, node, W1, att_src, att_dst, b1, W2, b2, W3, b3):
    raise NotImplementedError("write your pallas kernel here")


# speedup vs baseline: 3.0748x; 3.0748x over previous
import jax
import jax.numpy as jnp
from jax import lax
from jax.experimental import pallas as pl

N = 10000
E = 320000
F = 128
H = 10
OUT = 1024
HF = H * F

NP_ = 10240
ET = E + N
EB = 512
ETP = 330240
NEB = ETP // EB


def _tc_a_body(node_ref, w1_ref, atts_ref, attd_ref, xh_ref, as_ref, ad_ref):
    xh = jnp.dot(node_ref[...], w1_ref[...],
                 preferred_element_type=jnp.float32)
    xh_ref[...] = xh
    j = lax.broadcasted_iota(jnp.int32, (HF, 128), 0)
    l = lax.broadcasted_iota(jnp.int32, (HF, 128), 1)
    G = jnp.where(j // F == l, 1.0, 0.0).astype(jnp.float32)
    as_ref[...] = jnp.dot(xh * atts_ref[...], G,
                          preferred_element_type=jnp.float32)
    ad_ref[...] = jnp.dot(xh * attd_ref[...], G,
                          preferred_element_type=jnp.float32)


def _tc_a(node_p, W1, att_src, att_dst):
    B = 256
    return pl.pallas_call(
        _tc_a_body,
        grid=(NP_ // B,),
        in_specs=[
            pl.BlockSpec((B, F), lambda i: (i, 0)),
            pl.BlockSpec((F, HF), lambda i: (0, 0)),
            pl.BlockSpec((1, HF), lambda i: (0, 0)),
            pl.BlockSpec((1, HF), lambda i: (0, 0)),
        ],
        out_specs=[
            pl.BlockSpec((B, HF), lambda i: (i, 0)),
            pl.BlockSpec((B, 128), lambda i: (i, 0)),
            pl.BlockSpec((B, 128), lambda i: (i, 0)),
        ],
        out_shape=[
            jax.ShapeDtypeStruct((NP_, HF), jnp.float32),
            jax.ShapeDtypeStruct((NP_, 128), jnp.float32),
            jax.ShapeDtypeStruct((NP_, 128), jnp.float32),
        ],
    )(node_p, W1, att_src.reshape(1, HF), att_dst.reshape(1, HF))


def _edge_p_body(as_e_ref, ad_e_ref, p_ref):
    s = as_e_ref[...] + ad_e_ref[...]
    s = jnp.where(s >= 0.0, s, 0.2 * s)
    p_ref[...] = jnp.exp(s)


def _edge_p(as_e, ad_e):
    return pl.pallas_call(
        _edge_p_body,
        grid=(NEB,),
        in_specs=[
            pl.BlockSpec((EB, 128), lambda i: (i, 0)),
            pl.BlockSpec((EB, 128), lambda i: (i, 0)),
        ],
        out_specs=pl.BlockSpec((EB, 128), lambda i: (i, 0)),
        out_shape=jax.ShapeDtypeStruct((ETP, 128), jnp.float32),
    )(as_e, ad_e)


def _edge_msg_body(xh_e_ref, p_ref, den_ref, msg_ref):
    alpha = p_ref[...] / (den_ref[...] + 1e-16)
    l = lax.broadcasted_iota(jnp.int32, (128, HF), 0)
    j = lax.broadcasted_iota(jnp.int32, (128, HF), 1)
    R = jnp.where(l == j // F, 1.0, 0.0).astype(jnp.float32)
    a1280 = jnp.dot(alpha, R, preferred_element_type=jnp.float32)
    msg_ref[...] = xh_e_ref[...] * a1280


def _edge_msg(xh_e, p_e, den_e):
    return pl.pallas_call(
        _edge_msg_body,
        grid=(NEB,),
        in_specs=[
            pl.BlockSpec((EB, HF), lambda i: (i, 0)),
            pl.BlockSpec((EB, 128), lambda i: (i, 0)),
            pl.BlockSpec((EB, 128), lambda i: (i, 0)),
        ],
        out_specs=pl.BlockSpec((EB, HF), lambda i: (i, 0)),
        out_shape=jax.ShapeDtypeStruct((ETP, HF), jnp.float32),
    )(xh_e, p_e, den_e)


def _edge_gcn_body(xw_e_ref, ds_ref, dd_ref, msg_ref):
    norm = ds_ref[...] * dd_ref[...]
    msg_ref[...] = xw_e_ref[...] * norm


def _edge_gcn(xw_e, dinv_s, dinv_d):
    return pl.pallas_call(
        _edge_gcn_body,
        grid=(NEB,),
        in_specs=[
            pl.BlockSpec((EB, HF), lambda i: (i, 0)),
            pl.BlockSpec((EB, 1), lambda i: (i, 0)),
            pl.BlockSpec((EB, 1), lambda i: (i, 0)),
        ],
        out_specs=pl.BlockSpec((EB, HF), lambda i: (i, 0)),
        out_shape=jax.ShapeDtypeStruct((ETP, HF), jnp.float32),
    )(xw_e, dinv_s, dinv_d)


def _tc_j_body(gat_ref, b1_ref, w2_ref, deg_ref, xw_ref, dinv_ref):
    x2 = jnp.maximum(gat_ref[...] + b1_ref[...], 0.0)
    xw_ref[...] = jnp.dot(x2, w2_ref[...], preferred_element_type=jnp.float32)
    deg = deg_ref[...]
    dinv_ref[...] = jnp.where(deg > 0.0,
                              lax.rsqrt(jnp.maximum(deg, 1e-30)), 0.0)


def _tc_j(gat, b1, W2, deg_in):
    B = 256
    return pl.pallas_call(
        _tc_j_body,
        grid=(NP_ // B,),
        in_specs=[
            pl.BlockSpec((B, HF), lambda i: (i, 0)),
            pl.BlockSpec((1, HF), lambda i: (0, 0)),
            pl.BlockSpec((HF, HF), lambda i: (0, 0)),
            pl.BlockSpec((B, 1), lambda i: (i, 0)),
        ],
        out_specs=[
            pl.BlockSpec((B, HF), lambda i: (i, 0)),
            pl.BlockSpec((B, 1), lambda i: (i, 0)),
        ],
        out_shape=[
            jax.ShapeDtypeStruct((NP_, HF), jnp.float32),
            jax.ShapeDtypeStruct((NP_, 1), jnp.float32),
        ],
    )(gat, b1.reshape(1, HF), W2, deg_in)


def _tc_n_body(gcn_ref, b2_ref, w3_ref, b3_ref, out_ref):
    x3 = jnp.maximum(gcn_ref[...] + b2_ref[...], 0.0)
    out_ref[...] = jnp.dot(x3, w3_ref[...],
                           preferred_element_type=jnp.float32) + b3_ref[...]


def _tc_n(gcn, b2, W3, b3):
    B = 256
    return pl.pallas_call(
        _tc_n_body,
        grid=(NP_ // B,),
        in_specs=[
            pl.BlockSpec((B, HF), lambda i: (i, 0)),
            pl.BlockSpec((1, HF), lambda i: (0, 0)),
            pl.BlockSpec((HF, OUT), lambda i: (0, 0)),
            pl.BlockSpec((1, OUT), lambda i: (0, 0)),
        ],
        out_specs=pl.BlockSpec((B, OUT), lambda i: (i, 0)),
        out_shape=jax.ShapeDtypeStruct((NP_, OUT), jnp.float32),
    )(gcn, b2.reshape(1, HF), W3, b3.reshape(1, OUT))


def kernel(edge_index, node, W1, att_src, att_dst, b1, W2, b2, W3, b3):
    loop = jnp.arange(N, dtype=jnp.int32)
    src = jnp.concatenate([edge_index[0].astype(jnp.int32), loop])
    dst = jnp.concatenate([edge_index[1].astype(jnp.int32), loop])
    src_p = jnp.concatenate([src, jnp.zeros((ETP - ET,), jnp.int32)])
    dst_p = jnp.concatenate([dst, jnp.full((ETP - ET,), N, jnp.int32)])

    node_p = jnp.pad(node, ((0, NP_ - N), (0, 0)))
    xh, as128, ad128 = _tc_a(node_p, W1, att_src, att_dst)

    p = _edge_p(as128[src_p], ad128[dst_p])
    den_node = jax.ops.segment_sum(p, dst_p, num_segments=N + 1)
    msg = _edge_msg(xh[:N][src_p], p, den_node[dst_p])
    gat = jax.ops.segment_sum(msg, dst_p, num_segments=N + 1)[:N]

    deg_in = jnp.pad(den_node[:N, H:H + 1], ((0, NP_ - N), (0, 0)))
    gat_p = jnp.pad(gat, ((0, NP_ - N), (0, 0)))
    xw, dinv = _tc_j(gat_p, b1, W2, deg_in)

    dinv_n = dinv[:N]
    msg2 = _edge_gcn(xw[:N][src_p], dinv_n[src_p], dinv_n[dst_p])
    gcn = jax.ops.segment_sum(msg2, dst_p, num_segments=N + 1)[:N]

    gcn_p = jnp.pad(gcn, ((0, NP_ - N), (0, 0)))
    out = _tc_n(gcn_p, b2, W3, b3)
    return out[:N]
